# baseline (speedup 1.0000x reference)
import jax
import jax.numpy as jnp
from jax.experimental import pallas as pl
from jax.experimental.pallas import tpu as pltpu

J = 100
M = 20
N = J + M
NP = 128
SP = 104
BS = 256
H = 3
F0 = 16
ED = 128
BB = 64
FB = 16
NFB = ED // FB


def _lrelu(x, s):
    return jnp.maximum(x, s * x)


def _mm(x3, w):
    b, n, k = x3.shape
    y = jnp.dot(x3.reshape(b * n, k), w, preferred_element_type=jnp.float32)
    return y.reshape(b, n, -1)


def _gnn_block(nh_ref, nl_ref, nw_ref, np_ref, nn_ref, a_ref, t_ref,
               g_ref, bln_ref, w0_ref, al0_ref, ar0_ref, ae0_ref, we0_ref,
               b0_ref, w1_ref, al1_ref, ar1_ref, ae1_ref, we1_ref, b1_ref,
               h1_ref,
               w0p_s, wel0_s, wer0_s, few0_s, b0c_s,
               wel1_s, wer1_s, few1_s, b1c_s, eec_s):
    @pl.when(pl.program_id(0) == 0)
    def _():
        w0p_s[...] = jnp.concatenate(
            [w0_ref[...], jnp.zeros((3, H * F0), jnp.float32)], axis=0)
        for (w_ref, al_ref, ar_ref, ae_ref, we_ref, b_ref, Fh, Cpad, wel_s,
             wer_s, few_s, bc_s, ebase) in (
                (w0_ref, al0_ref, ar0_ref, ae0_ref, we0_ref, b0_ref, F0, 8,
                 wel0_s, wer0_s, few0_s, b0c_s, 0),
                (w1_ref, al1_ref, ar1_ref, ae1_ref, we1_ref, b1_ref, ED, F0,
                 wel1_s, wer1_s, few1_s, b1c_s, H)):
            C = w_ref.shape[0]
            zpad = jnp.zeros((max(Cpad - C, 1), 1), jnp.float32)
            for h in range(H):
                wsl = w_ref[:, h * Fh:(h + 1) * Fh]
                wel = jnp.sum(wsl * al_ref[h:h + 1, :], axis=1, keepdims=True)
                wer = jnp.sum(wsl * ar_ref[h:h + 1, :], axis=1, keepdims=True)
                wesl = we_ref[:, h * Fh:(h + 1) * Fh]
                if Cpad > C:
                    wel = jnp.concatenate([wel, zpad], axis=0)
                    wer = jnp.concatenate([wer, zpad], axis=0)
                wel_s[:, h:h + 1] = wel
                wer_s[:, h:h + 1] = wer
                few_s[:, h:h + 1] = jnp.swapaxes(wesl, 0, 1)
                bc_s[:, h:h + 1] = jnp.swapaxes(
                    b_ref[:, h * Fh:(h + 1) * Fh], 0, 1)
                eec_s[ebase + h] = jnp.sum(wesl * ae_ref[h:h + 1, :])

    zlane = jnp.zeros((BB, NP - J), jnp.float32)
    lane = jax.lax.broadcasted_iota(jnp.int32, (BB, NP), 1)
    mJ = lane < J
    rows = [jnp.concatenate([nh_ref[...], zlane], axis=1),
            jnp.concatenate([nl_ref[...], zlane], axis=1),
            jnp.where(mJ, nw_ref[...], 0.0),
            jnp.where(mJ, np_ref[...], 0.0),
            jnp.where(mJ, nn_ref[...], 0.0)]
    XT = jnp.stack(rows, axis=1)

    mu = jnp.sum(XT, axis=1, keepdims=True) * (1.0 / 5.0)
    d = XT - mu
    var = jnp.sum(d * d, axis=1, keepdims=True) * (1.0 / 5.0)
    zn = d * jax.lax.rsqrt(var + 1e-5)
    XnT = jnp.concatenate(
        [zn[:, c:c + 1, :] * g_ref[c] + bln_ref[c] for c in range(5)]
        + [jnp.zeros((BB, 3, NP), jnp.float32)], axis=1)

    G = a_ref[...]
    Gp = jnp.concatenate([G, jnp.zeros((BB, SP - J, N), jnp.float32)], axis=1)
    Gp = jnp.concatenate([Gp, jnp.zeros((BB, SP, NP - N), jnp.float32)], axis=2)
    Ab = Gp > 0
    Tr = t_ref[...]
    Tm = jnp.concatenate([Tr, jnp.zeros((BB, SP - J, J), jnp.float32)], axis=1)
    Tm = jnp.concatenate([Tm, jnp.zeros((BB, SP, NP - J), jnp.float32)], axis=2)

    def gat_T(ft, el3, srcT, wer_s, few_s, bc_s, Fh, ebase):
        C = srcT.shape[1]
        acc = None
        for h in range(H):
            fth = ft[:, :, h * Fh:(h + 1) * Fh]
            el = el3[:, :, h:h + 1]
            er = jnp.sum(srcT * wer_s[:, h:h + 1].reshape(1, C, 1),
                         axis=1, keepdims=True)
            eec = eec_s[ebase + h]
            logits = el + er + Tm * eec
            logits = _lrelu(logits, 0.2)
            logits = jnp.where(Ab, logits, -1e9)
            mx = jnp.max(logits, axis=1, keepdims=True)
            ex = jnp.exp(logits - mx)
            den = jnp.sum(ex, axis=1, keepdims=True)
            rden = jnp.where(mx > -1e8, 1.0 / den, 0.0)
            outT = jax.lax.dot_general(
                fth, ex, (((1,), (1,)), ((0,), (0,))),
                preferred_element_type=jnp.float32)
            eaggT = jnp.sum(ex * Tm, axis=1, keepdims=True)
            fewcol = few_s[:, h:h + 1].reshape(1, Fh, 1)
            bcol = bc_s[:, h:h + 1].reshape(1, Fh, 1)
            hh = _lrelu((outT + eaggT * fewcol) * rden + bcol, 0.01)
            acc = hh if acc is None else acc + hh
        return acc * (1.0 / H)

    Xn = jnp.swapaxes(XnT[:, :, :SP], 1, 2)
    ft0 = _mm(Xn, w0p_s[...])
    el03 = _mm(Xn, wel0_s[...])
    h0T = gat_T(ft0, el03, XnT, wer0_s, few0_s, b0c_s, F0, 0)

    h0 = jnp.swapaxes(h0T[:, :, :SP], 1, 2)
    ft1 = _mm(h0, w1_ref[...])
    el13 = _mm(h0, wel1_s[...])
    h1_ref[...] = gat_T(ft1, el13, h0T, wer1_s, few1_s, b1c_s, ED, H)


def _fin_block(h_ref, w_ref, b_ref, o_ref):
    k = pl.program_id(0)
    hblk = h_ref[...]
    wblk = w_ref[...]
    part = None
    for ff in range(FB):
        p = jnp.dot(hblk[:, ff, :N], wblk[:, ff, :],
                    preferred_element_type=jnp.float32)
        part = p if part is None else part + p

    @pl.when(k == 0)
    def _():
        o_ref[...] = part

    @pl.when(k > 0)
    def _():
        o_ref[...] += part

    @pl.when(k == NFB - 1)
    def _():
        o_ref[...] = _lrelu(o_ref[...] + b_ref[...], 0.01)


def kernel(Graph, norm_h, norm_L, norm_W, norm_P, norm_N, T, ln_g, ln_b,
           W0, We0, al0, ar0, ae0, b0, W1, We1, al1, ar1, ae1, b1, Wl, bl):
    G3 = Graph.reshape(BS, J, N)
    b0r = b0.reshape(1, H * F0)
    b1r = b1.reshape(1, H * ED)
    Wl3 = Wl.reshape(N, ED, ED)
    blr = bl.reshape(1, ED)

    rep2 = lambda i: (0, 0)
    smem = pl.BlockSpec(memory_space=pltpu.SMEM)
    h1t = pl.pallas_call(
        _gnn_block,
        grid=(BS // BB,),
        in_specs=[
            pl.BlockSpec((BB, J), lambda i: (i, 0)),
            pl.BlockSpec((BB, J), lambda i: (i, 0)),
            pl.BlockSpec((BB, 1), lambda i: (i, 0)),
            pl.BlockSpec((BB, 1), lambda i: (i, 0)),
            pl.BlockSpec((BB, 1), lambda i: (i, 0)),
            pl.BlockSpec((BB, J, N), lambda i: (i, 0, 0)),
            pl.BlockSpec((BB, J, J), lambda i: (i, 0, 0)),
            smem,
            smem,
            pl.BlockSpec((5, H * F0), rep2),
            pl.BlockSpec((H, F0), rep2),
            pl.BlockSpec((H, F0), rep2),
            pl.BlockSpec((H, F0), rep2),
            pl.BlockSpec((1, H * F0), rep2),
            pl.BlockSpec((1, H * F0), rep2),
            pl.BlockSpec((F0, H * ED), rep2),
            pl.BlockSpec((H, ED), rep2),
            pl.BlockSpec((H, ED), rep2),
            pl.BlockSpec((H, ED), rep2),
            pl.BlockSpec((1, H * ED), rep2),
            pl.BlockSpec((1, H * ED), rep2),
        ],
        out_specs=pl.BlockSpec((BB, ED, NP), lambda i: (i, 0, 0)),
        out_shape=jax.ShapeDtypeStruct((BS, ED, NP), jnp.float32),
        scratch_shapes=[
            pltpu.VMEM((8, H * F0), jnp.float32),
            pltpu.VMEM((8, H), jnp.float32),
            pltpu.VMEM((8, H), jnp.float32),
            pltpu.VMEM((F0, H), jnp.float32),
            pltpu.VMEM((F0, H), jnp.float32),
            pltpu.VMEM((F0, H), jnp.float32),
            pltpu.VMEM((F0, H), jnp.float32),
            pltpu.VMEM((ED, H), jnp.float32),
            pltpu.VMEM((ED, H), jnp.float32),
            pltpu.SMEM((2 * H,), jnp.float32),
        ],
    )(norm_h, norm_L, norm_W, norm_P, norm_N, G3, T, ln_g, ln_b,
      W0, al0, ar0, ae0, We0, b0r, W1, al1, ar1, ae1, We1, b1r)

    out = pl.pallas_call(
        _fin_block,
        grid=(NFB,),
        in_specs=[
            pl.BlockSpec((BS, FB, NP), lambda k: (0, k, 0)),
            pl.BlockSpec((N, FB, ED), lambda k: (0, k, 0)),
            pl.BlockSpec((1, ED), lambda k: (0, 0)),
        ],
        out_specs=pl.BlockSpec((BS, ED), lambda k: (0, 0)),
        out_shape=jax.ShapeDtypeStruct((BS, ED), jnp.float32),
    )(h1t, Wl3, blr)
    return out


# speedup vs baseline: 1.0111x; 1.0111x over previous
import jax
import jax.numpy as jnp
from jax.experimental import pallas as pl
from jax.experimental.pallas import tpu as pltpu

J = 100
M = 20
N = J + M
NP = 128
SP = 104
BS = 256
H = 3
F0 = 16
ED = 128
BB = 32
FB = 32
NFB = ED // FB


def _lrelu(x, s):
    return jnp.maximum(x, s * x)


def _mm(x3, w):
    b, n, k = x3.shape
    y = jnp.dot(x3.reshape(b * n, k), w, preferred_element_type=jnp.float32)
    return y.reshape(b, n, -1)


def _gnn_block(nh_ref, nl_ref, nw_ref, np_ref, nn_ref, a_ref, t_ref,
               g_ref, bln_ref, w0_ref, al0_ref, ar0_ref, ae0_ref, we0_ref,
               b0_ref, w1_ref, al1_ref, ar1_ref, ae1_ref, we1_ref, b1_ref,
               h1_ref,
               w0p_s, wel0_s, wer0_s, few0_s, b0c_s,
               wel1_s, wer1_s, few1_s, b1c_s, eec_s):
    @pl.when(pl.program_id(0) == 0)
    def _():
        w0p_s[...] = jnp.concatenate(
            [w0_ref[...], jnp.zeros((3, H * F0), jnp.float32)], axis=0)
        for (w_ref, al_ref, ar_ref, ae_ref, we_ref, b_ref, Fh, Cpad, wel_s,
             wer_s, few_s, bc_s, ebase) in (
                (w0_ref, al0_ref, ar0_ref, ae0_ref, we0_ref, b0_ref, F0, 8,
                 wel0_s, wer0_s, few0_s, b0c_s, 0),
                (w1_ref, al1_ref, ar1_ref, ae1_ref, we1_ref, b1_ref, ED, F0,
                 wel1_s, wer1_s, few1_s, b1c_s, H)):
            C = w_ref.shape[0]
            zpad = jnp.zeros((max(Cpad - C, 1), 1), jnp.float32)
            for h in range(H):
                wsl = w_ref[:, h * Fh:(h + 1) * Fh]
                wel = jnp.sum(wsl * al_ref[h:h + 1, :], axis=1, keepdims=True)
                wer = jnp.sum(wsl * ar_ref[h:h + 1, :], axis=1, keepdims=True)
                wesl = we_ref[:, h * Fh:(h + 1) * Fh]
                if Cpad > C:
                    wel = jnp.concatenate([wel, zpad], axis=0)
                    wer = jnp.concatenate([wer, zpad], axis=0)
                wel_s[:, h:h + 1] = wel
                wer_s[:, h:h + 1] = wer
                few_s[:, h:h + 1] = jnp.swapaxes(wesl, 0, 1)
                bc_s[:, h:h + 1] = jnp.swapaxes(
                    b_ref[:, h * Fh:(h + 1) * Fh], 0, 1)
                eec_s[ebase + h] = jnp.sum(wesl * ae_ref[h:h + 1, :])

    zlane = jnp.zeros((BB, NP - J), jnp.float32)
    lane = jax.lax.broadcasted_iota(jnp.int32, (BB, NP), 1)
    mJ = lane < J
    rows = [jnp.concatenate([nh_ref[...], zlane], axis=1),
            jnp.concatenate([nl_ref[...], zlane], axis=1),
            jnp.where(mJ, nw_ref[...], 0.0),
            jnp.where(mJ, np_ref[...], 0.0),
            jnp.where(mJ, nn_ref[...], 0.0)]
    XT = jnp.stack(rows, axis=1)

    mu = jnp.sum(XT, axis=1, keepdims=True) * (1.0 / 5.0)
    d = XT - mu
    var = jnp.sum(d * d, axis=1, keepdims=True) * (1.0 / 5.0)
    zn = d * jax.lax.rsqrt(var + 1e-5)
    XnT = jnp.concatenate(
        [zn[:, c:c + 1, :] * g_ref[c] + bln_ref[c] for c in range(5)]
        + [jnp.zeros((BB, 3, NP), jnp.float32)], axis=1)

    G = a_ref[...]
    Gp = jnp.concatenate([G, jnp.zeros((BB, SP - J, N), jnp.float32)], axis=1)
    Gp = jnp.concatenate([Gp, jnp.zeros((BB, SP, NP - N), jnp.float32)], axis=2)
    Ab = Gp > 0
    Tr = t_ref[...]
    Tm = jnp.concatenate([Tr, jnp.zeros((BB, SP - J, J), jnp.float32)], axis=1)
    Tm = jnp.concatenate([Tm, jnp.zeros((BB, SP, NP - J), jnp.float32)], axis=2)

    def gat_T(ft, el3, srcT, wer_s, few_s, bc_s, Fh, ebase):
        C = srcT.shape[1]
        acc = None
        for h in range(H):
            fth = ft[:, :, h * Fh:(h + 1) * Fh]
            el = el3[:, :, h:h + 1]
            er = jnp.sum(srcT * wer_s[:, h:h + 1].reshape(1, C, 1),
                         axis=1, keepdims=True)
            eec = eec_s[ebase + h]
            logits = el + er + Tm * eec
            logits = _lrelu(logits, 0.2)
            logits = jnp.where(Ab, logits, -1e9)
            mx = jnp.max(logits, axis=1, keepdims=True)
            ex = jnp.exp(logits - mx)
            den = jnp.sum(ex, axis=1, keepdims=True)
            rden = jnp.where(mx > -1e8, 1.0 / den, 0.0)
            outT = jax.lax.dot_general(
                fth, ex, (((1,), (1,)), ((0,), (0,))),
                preferred_element_type=jnp.float32)
            eaggT = jnp.sum(ex * Tm, axis=1, keepdims=True)
            fewcol = few_s[:, h:h + 1].reshape(1, Fh, 1)
            bcol = bc_s[:, h:h + 1].reshape(1, Fh, 1)
            hh = _lrelu((outT + eaggT * fewcol) * rden + bcol, 0.01)
            acc = hh if acc is None else acc + hh
        return acc * (1.0 / H)

    Xn = jnp.swapaxes(XnT[:, :, :SP], 1, 2)
    ft0 = _mm(Xn, w0p_s[...])
    el03 = _mm(Xn, wel0_s[...])
    h0T = gat_T(ft0, el03, XnT, wer0_s, few0_s, b0c_s, F0, 0)

    h0 = jnp.swapaxes(h0T[:, :, :SP], 1, 2)
    ft1 = _mm(h0, w1_ref[...])
    el13 = _mm(h0, wel1_s[...])
    h1_ref[...] = gat_T(ft1, el13, h0T, wer1_s, few1_s, b1c_s, ED, H)


def _fin_block(h_ref, w_ref, b_ref, o_ref):
    k = pl.program_id(0)
    hblk = h_ref[...]
    wblk = w_ref[...]
    part = None
    for ff in range(FB):
        p = jnp.dot(hblk[:, ff, :N], wblk[:, ff, :],
                    preferred_element_type=jnp.float32)
        part = p if part is None else part + p

    @pl.when(k == 0)
    def _():
        o_ref[...] = part

    @pl.when(k > 0)
    def _():
        o_ref[...] += part

    @pl.when(k == NFB - 1)
    def _():
        o_ref[...] = _lrelu(o_ref[...] + b_ref[...], 0.01)


def kernel(Graph, norm_h, norm_L, norm_W, norm_P, norm_N, T, ln_g, ln_b,
           W0, We0, al0, ar0, ae0, b0, W1, We1, al1, ar1, ae1, b1, Wl, bl):
    G3 = Graph.reshape(BS, J, N)
    b0r = b0.reshape(1, H * F0)
    b1r = b1.reshape(1, H * ED)
    Wl3 = Wl.reshape(N, ED, ED)
    blr = bl.reshape(1, ED)

    rep2 = lambda i: (0, 0)
    smem = pl.BlockSpec(memory_space=pltpu.SMEM)
    h1t = pl.pallas_call(
        _gnn_block,
        grid=(BS // BB,),
        in_specs=[
            pl.BlockSpec((BB, J), lambda i: (i, 0)),
            pl.BlockSpec((BB, J), lambda i: (i, 0)),
            pl.BlockSpec((BB, 1), lambda i: (i, 0)),
            pl.BlockSpec((BB, 1), lambda i: (i, 0)),
            pl.BlockSpec((BB, 1), lambda i: (i, 0)),
            pl.BlockSpec((BB, J, N), lambda i: (i, 0, 0)),
            pl.BlockSpec((BB, J, J), lambda i: (i, 0, 0)),
            smem,
            smem,
            pl.BlockSpec((5, H * F0), rep2),
            pl.BlockSpec((H, F0), rep2),
            pl.BlockSpec((H, F0), rep2),
            pl.BlockSpec((H, F0), rep2),
            pl.BlockSpec((1, H * F0), rep2),
            pl.BlockSpec((1, H * F0), rep2),
            pl.BlockSpec((F0, H * ED), rep2),
            pl.BlockSpec((H, ED), rep2),
            pl.BlockSpec((H, ED), rep2),
            pl.BlockSpec((H, ED), rep2),
            pl.BlockSpec((1, H * ED), rep2),
            pl.BlockSpec((1, H * ED), rep2),
        ],
        out_specs=pl.BlockSpec((BB, ED, NP), lambda i: (i, 0, 0)),
        out_shape=jax.ShapeDtypeStruct((BS, ED, NP), jnp.float32),
        scratch_shapes=[
            pltpu.VMEM((8, H * F0), jnp.float32),
            pltpu.VMEM((8, H), jnp.float32),
            pltpu.VMEM((8, H), jnp.float32),
            pltpu.VMEM((F0, H), jnp.float32),
            pltpu.VMEM((F0, H), jnp.float32),
            pltpu.VMEM((F0, H), jnp.float32),
            pltpu.VMEM((F0, H), jnp.float32),
            pltpu.VMEM((ED, H), jnp.float32),
            pltpu.VMEM((ED, H), jnp.float32),
            pltpu.SMEM((2 * H,), jnp.float32),
        ],
    )(norm_h, norm_L, norm_W, norm_P, norm_N, G3, T, ln_g, ln_b,
      W0, al0, ar0, ae0, We0, b0r, W1, al1, ar1, ae1, We1, b1r)

    out = pl.pallas_call(
        _fin_block,
        grid=(NFB,),
        in_specs=[
            pl.BlockSpec((BS, FB, NP), lambda k: (0, k, 0)),
            pl.BlockSpec((N, FB, ED), lambda k: (0, k, 0)),
            pl.BlockSpec((1, ED), lambda k: (0, 0)),
        ],
        out_specs=pl.BlockSpec((BS, ED), lambda k: (0, 0)),
        out_shape=jax.ShapeDtypeStruct((BS, ED), jnp.float32),
    )(h1t, Wl3, blr)
    return out


# speedup vs baseline: 1.0141x; 1.0030x over previous
import jax
import jax.numpy as jnp
from jax.experimental import pallas as pl
from jax.experimental.pallas import tpu as pltpu

J = 100
M = 20
N = J + M
NP = 128
SP = 104
BS = 256
H = 3
F0 = 16
ED = 128
BB = 32
FB = 16
NFB = ED // FB


def _lrelu(x, s):
    return jnp.maximum(x, s * x)


def _mm(x3, w):
    b, n, k = x3.shape
    y = jnp.dot(x3.reshape(b * n, k), w, preferred_element_type=jnp.float32)
    return y.reshape(b, n, -1)


def _gnn_block(nh_ref, nl_ref, nw_ref, np_ref, nn_ref, a_ref, t_ref,
               g_ref, bln_ref, w0_ref, al0_ref, ar0_ref, ae0_ref, we0_ref,
               b0_ref, w1_ref, al1_ref, ar1_ref, ae1_ref, we1_ref, b1_ref,
               h1_ref,
               w0p_s, wel0_s, wer0_s, few0_s, b0c_s,
               wel1_s, wer1_s, few1_s, b1c_s, eec_s):
    @pl.when(pl.program_id(0) == 0)
    def _():
        w0p_s[...] = jnp.concatenate(
            [w0_ref[...], jnp.zeros((3, H * F0), jnp.float32)], axis=0)
        for (w_ref, al_ref, ar_ref, ae_ref, we_ref, b_ref, Fh, Cpad, wel_s,
             wer_s, few_s, bc_s, ebase) in (
                (w0_ref, al0_ref, ar0_ref, ae0_ref, we0_ref, b0_ref, F0, 8,
                 wel0_s, wer0_s, few0_s, b0c_s, 0),
                (w1_ref, al1_ref, ar1_ref, ae1_ref, we1_ref, b1_ref, ED, F0,
                 wel1_s, wer1_s, few1_s, b1c_s, H)):
            C = w_ref.shape[0]
            zpad = jnp.zeros((max(Cpad - C, 1), 1), jnp.float32)
            for h in range(H):
                wsl = w_ref[:, h * Fh:(h + 1) * Fh]
                wel = jnp.sum(wsl * al_ref[h:h + 1, :], axis=1, keepdims=True)
                wer = jnp.sum(wsl * ar_ref[h:h + 1, :], axis=1, keepdims=True)
                wesl = we_ref[:, h * Fh:(h + 1) * Fh]
                if Cpad > C:
                    wel = jnp.concatenate([wel, zpad], axis=0)
                    wer = jnp.concatenate([wer, zpad], axis=0)
                wel_s[:, h:h + 1] = wel
                wer_s[:, h:h + 1] = wer
                few_s[:, h:h + 1] = jnp.swapaxes(wesl, 0, 1)
                bc_s[:, h:h + 1] = jnp.swapaxes(
                    b_ref[:, h * Fh:(h + 1) * Fh], 0, 1)
                eec_s[ebase + h] = jnp.sum(wesl * ae_ref[h:h + 1, :])

    zlane = jnp.zeros((BB, NP - J), jnp.float32)
    lane = jax.lax.broadcasted_iota(jnp.int32, (BB, NP), 1)
    mJ = lane < J
    rows = [jnp.concatenate([nh_ref[...], zlane], axis=1),
            jnp.concatenate([nl_ref[...], zlane], axis=1),
            jnp.where(mJ, nw_ref[...], 0.0),
            jnp.where(mJ, np_ref[...], 0.0),
            jnp.where(mJ, nn_ref[...], 0.0)]
    XT = jnp.stack(rows, axis=1)

    mu = jnp.sum(XT, axis=1, keepdims=True) * (1.0 / 5.0)
    d = XT - mu
    var = jnp.sum(d * d, axis=1, keepdims=True) * (1.0 / 5.0)
    zn = d * jax.lax.rsqrt(var + 1e-5)
    XnT = jnp.concatenate(
        [zn[:, c:c + 1, :] * g_ref[c] + bln_ref[c] for c in range(5)]
        + [jnp.zeros((BB, 3, NP), jnp.float32)], axis=1)

    G = a_ref[...]
    Gp = jnp.concatenate([G, jnp.zeros((BB, SP - J, N), jnp.float32)], axis=1)
    Gp = jnp.concatenate([Gp, jnp.zeros((BB, SP, NP - N), jnp.float32)], axis=2)
    Ab = Gp > 0
    Tr = t_ref[...]
    Tm = jnp.concatenate([Tr, jnp.zeros((BB, SP - J, J), jnp.float32)], axis=1)
    Tm = jnp.concatenate([Tm, jnp.zeros((BB, SP, NP - J), jnp.float32)], axis=2)

    def gat_T(ft, el3, srcT, wer_s, few_s, bc_s, Fh, ebase):
        C = srcT.shape[1]
        acc = None
        for h in range(H):
            fth = ft[:, :, h * Fh:(h + 1) * Fh]
            el = el3[:, :, h:h + 1]
            er = jnp.sum(srcT * wer_s[:, h:h + 1].reshape(1, C, 1),
                         axis=1, keepdims=True)
            eec = eec_s[ebase + h]
            logits = el + er + Tm * eec
            logits = _lrelu(logits, 0.2)
            logits = jnp.where(Ab, logits, -1e9)
            mx = jnp.max(logits, axis=1, keepdims=True)
            ex = jnp.exp(logits - mx)
            den = jnp.sum(ex, axis=1, keepdims=True)
            rden = jnp.where(mx > -1e8, 1.0 / den, 0.0)
            outT = jax.lax.dot_general(
                fth, ex, (((1,), (1,)), ((0,), (0,))),
                preferred_element_type=jnp.float32)
            eaggT = jnp.sum(ex * Tm, axis=1, keepdims=True)
            fewcol = few_s[:, h:h + 1].reshape(1, Fh, 1)
            bcol = bc_s[:, h:h + 1].reshape(1, Fh, 1)
            hh = _lrelu((outT + eaggT * fewcol) * rden + bcol, 0.01)
            acc = hh if acc is None else acc + hh
        return acc * (1.0 / H)

    Xn = jnp.swapaxes(XnT[:, :, :SP], 1, 2)
    ft0 = _mm(Xn, w0p_s[...])
    el03 = _mm(Xn, wel0_s[...])
    h0T = gat_T(ft0, el03, XnT, wer0_s, few0_s, b0c_s, F0, 0)

    h0 = jnp.swapaxes(h0T[:, :, :SP], 1, 2)
    ft1 = _mm(h0, w1_ref[...])
    el13 = _mm(h0, wel1_s[...])
    h1_ref[...] = gat_T(ft1, el13, h0T, wer1_s, few1_s, b1c_s, ED, H)


def _fin_block(h_ref, w_ref, b_ref, o_ref):
    k = pl.program_id(0)
    hblk = h_ref[...]
    wblk = w_ref[...]
    part = None
    for ff in range(FB):
        p = jnp.dot(hblk[:, ff, :N], wblk[:, ff, :],
                    preferred_element_type=jnp.float32)
        part = p if part is None else part + p

    @pl.when(k == 0)
    def _():
        o_ref[...] = part

    @pl.when(k > 0)
    def _():
        o_ref[...] += part

    @pl.when(k == NFB - 1)
    def _():
        o_ref[...] = _lrelu(o_ref[...] + b_ref[...], 0.01)


def kernel(Graph, norm_h, norm_L, norm_W, norm_P, norm_N, T, ln_g, ln_b,
           W0, We0, al0, ar0, ae0, b0, W1, We1, al1, ar1, ae1, b1, Wl, bl):
    G3 = Graph.reshape(BS, J, N)
    b0r = b0.reshape(1, H * F0)
    b1r = b1.reshape(1, H * ED)
    Wl3 = Wl.reshape(N, ED, ED)
    blr = bl.reshape(1, ED)

    rep2 = lambda i: (0, 0)
    smem = pl.BlockSpec(memory_space=pltpu.SMEM)
    h1t = pl.pallas_call(
        _gnn_block,
        grid=(BS // BB,),
        in_specs=[
            pl.BlockSpec((BB, J), lambda i: (i, 0)),
            pl.BlockSpec((BB, J), lambda i: (i, 0)),
            pl.BlockSpec((BB, 1), lambda i: (i, 0)),
            pl.BlockSpec((BB, 1), lambda i: (i, 0)),
            pl.BlockSpec((BB, 1), lambda i: (i, 0)),
            pl.BlockSpec((BB, J, N), lambda i: (i, 0, 0)),
            pl.BlockSpec((BB, J, J), lambda i: (i, 0, 0)),
            smem,
            smem,
            pl.BlockSpec((5, H * F0), rep2),
            pl.BlockSpec((H, F0), rep2),
            pl.BlockSpec((H, F0), rep2),
            pl.BlockSpec((H, F0), rep2),
            pl.BlockSpec((1, H * F0), rep2),
            pl.BlockSpec((1, H * F0), rep2),
            pl.BlockSpec((F0, H * ED), rep2),
            pl.BlockSpec((H, ED), rep2),
            pl.BlockSpec((H, ED), rep2),
            pl.BlockSpec((H, ED), rep2),
            pl.BlockSpec((1, H * ED), rep2),
            pl.BlockSpec((1, H * ED), rep2),
        ],
        out_specs=pl.BlockSpec((BB, ED, NP), lambda i: (i, 0, 0)),
        out_shape=jax.ShapeDtypeStruct((BS, ED, NP), jnp.float32),
        scratch_shapes=[
            pltpu.VMEM((8, H * F0), jnp.float32),
            pltpu.VMEM((8, H), jnp.float32),
            pltpu.VMEM((8, H), jnp.float32),
            pltpu.VMEM((F0, H), jnp.float32),
            pltpu.VMEM((F0, H), jnp.float32),
            pltpu.VMEM((F0, H), jnp.float32),
            pltpu.VMEM((F0, H), jnp.float32),
            pltpu.VMEM((ED, H), jnp.float32),
            pltpu.VMEM((ED, H), jnp.float32),
            pltpu.SMEM((2 * H,), jnp.float32),
        ],
    )(norm_h, norm_L, norm_W, norm_P, norm_N, G3, T, ln_g, ln_b,
      W0, al0, ar0, ae0, We0, b0r, W1, al1, ar1, ae1, We1, b1r)

    out = pl.pallas_call(
        _fin_block,
        grid=(NFB,),
        in_specs=[
            pl.BlockSpec((BS, FB, NP), lambda k: (0, k, 0)),
            pl.BlockSpec((N, FB, ED), lambda k: (0, k, 0)),
            pl.BlockSpec((1, ED), lambda k: (0, 0)),
        ],
        out_specs=pl.BlockSpec((BS, ED), lambda k: (0, 0)),
        out_shape=jax.ShapeDtypeStruct((BS, ED), jnp.float32),
    )(h1t, Wl3, blr)
    return out
